# baseline (speedup 1.0000x reference)
import jax
import jax.numpy as jnp
from jax import lax
from jax.experimental import pallas as pl
from jax.experimental.pallas import tpu as pltpu
from jax.experimental.pallas import tpu_sc as plsc

N = 10000
D = 128
NUM_EDGES = 320000

NC = 2
NS = 16
NW = NC * NS

CHUNK = 128
CHUNKS = 80
HALF = CHUNKS // 2
E_PAD = NW * CHUNKS * CHUNK
N_PAD = 10240
STRIPE = N_PAD // NS
DEG_W = 16

_MESH = plsc.VectorSubcoreMesh(core_axis_name="c", subcore_axis_name="s")


def _zero_vmem(ref, nwords):
    cols = ref.shape[-1]
    per_row = cols // 16

    def body(i, _):
        r = lax.div(i, per_row)
        c = lax.rem(i, per_row) * 16
        ref[r, pl.ds(c, 16)] = jnp.zeros((16,), jnp.float32)
        return 0

    lax.fori_loop(0, nwords // 16, body, 0)


def _fill_ones(ref):
    rows, cols = ref.shape
    per_row = cols // 16

    def body(i, _):
        r = lax.div(i, per_row)
        c = lax.rem(i, per_row) * 16
        ref[r, pl.ds(c, 16)] = jnp.ones((16,), jnp.float32)
        return 0

    lax.fori_loop(0, rows * per_row, body, 0)


def _unpack_lo(pk, j, buf, b):
    def f(k, _):
        v = pk[j, pl.ds(k * 16, 16)]
        buf[b, pl.ds(k * 16, 16)] = jnp.bitwise_and(v, 0xFFFF)
        return 0
    lax.fori_loop(0, CHUNK // 16, f, 0)


def _unpack_hi(pk, j, buf, b):
    def f(k, _):
        v = pk[j, pl.ds(k * 16, 16)]
        buf[b, pl.ds(k * 16, 16)] = jnp.right_shift(v, 16)
        return 0
    lax.fori_loop(0, CHUNK // 16, f, 0)


def _make_agg():
    out_type = jax.ShapeDtypeStruct((NC, N_PAD, D), jnp.float32)
    scratch = [
        pltpu.VMEM((CHUNKS, CHUNK), jnp.int32),
        pltpu.VMEM((2, CHUNK), jnp.int32),
        pltpu.VMEM((2, CHUNK), jnp.int32),
        pltpu.VMEM((2, CHUNK, D), jnp.float32),
        pltpu.VMEM_SHARED((N_PAD, D), jnp.float32),
        pltpu.SemaphoreType.DMA,
        pltpu.SemaphoreType.DMA,
        pltpu.SemaphoreType.DMA,
        pltpu.SemaphoreType.DMA,
    ]

    def body(t_hbm, pk_hbm, out_hbm, pk, sbuf, dbuf, rows, acc,
             gsem0, gsem1, ssem0, ssem1):
        cid = lax.axis_index("c")
        sid = lax.axis_index("s")
        w = cid * NS + sid

        pltpu.sync_copy(pk_hbm.at[w], pk)

        _zero_vmem(rows.at[0], CHUNK * D)
        for k in range(STRIPE // CHUNK):
            pltpu.sync_copy(rows.at[0], acc.at[pl.ds(sid * STRIPE + k * CHUNK, CHUNK)])
        plsc.subcore_barrier()

        _unpack_lo(pk, 0, sbuf, 0)
        pltpu.async_copy(t_hbm.at[sbuf.at[0]], rows.at[0], gsem0)
        _unpack_lo(pk, 1, sbuf, 1)
        pltpu.async_copy(t_hbm.at[sbuf.at[1]], rows.at[1], gsem1)

        def step(i, _):
            j0 = 2 * i
            j1 = j0 + 1
            pltpu.make_async_copy(t_hbm.at[sbuf.at[0]], rows.at[0], gsem0).wait()
            _unpack_hi(pk, j0, dbuf, 0)
            pltpu.async_copy(rows.at[0], acc.at[dbuf.at[0]], ssem0, add=True)
            pltpu.make_async_copy(t_hbm.at[sbuf.at[1]], rows.at[1], gsem1).wait()
            _unpack_hi(pk, j1, dbuf, 1)
            pltpu.async_copy(rows.at[1], acc.at[dbuf.at[1]], ssem1, add=True)

            @pl.when(i + 1 < HALF)
            def _():
                pltpu.make_async_copy(rows.at[0], acc.at[dbuf.at[0]], ssem0).wait()
                _unpack_lo(pk, j0 + 2, sbuf, 0)
                pltpu.async_copy(t_hbm.at[sbuf.at[0]], rows.at[0], gsem0)
                pltpu.make_async_copy(rows.at[1], acc.at[dbuf.at[1]], ssem1).wait()
                _unpack_lo(pk, j1 + 2, sbuf, 1)
                pltpu.async_copy(t_hbm.at[sbuf.at[1]], rows.at[1], gsem1)
            return 0

        lax.fori_loop(0, HALF, step, 0)
        pltpu.make_async_copy(rows.at[0], acc.at[dbuf.at[0]], ssem0).wait()
        pltpu.make_async_copy(rows.at[1], acc.at[dbuf.at[1]], ssem1).wait()
        plsc.subcore_barrier()

        pltpu.sync_copy(acc.at[pl.ds(sid * STRIPE, STRIPE)],
                        out_hbm.at[cid, pl.ds(sid * STRIPE, STRIPE)])

    return pl.kernel(body, out_type=out_type, mesh=_MESH,
                     scratch_types=tuple(scratch),
                     compiler_params=pltpu.CompilerParams(
                         use_tc_tiling_on_sc=False))


def _make_deg():
    out_type = jax.ShapeDtypeStruct((NC, N_PAD, DEG_W), jnp.float32)
    scratch = [
        pltpu.VMEM((CHUNKS, CHUNK), jnp.int32),
        pltpu.VMEM((1, CHUNK), jnp.int32),
        pltpu.VMEM((CHUNK, DEG_W), jnp.float32),
        pltpu.VMEM((STRIPE, DEG_W), jnp.float32),
        pltpu.VMEM_SHARED((N_PAD, DEG_W), jnp.float32),
    ]

    def body(pk_hbm, deg_hbm, pk, dbuf, ones_v, degz, dacc):
        cid = lax.axis_index("c")
        sid = lax.axis_index("s")
        w = cid * NS + sid

        pltpu.sync_copy(pk_hbm.at[w], pk)
        _fill_ones(ones_v)
        _zero_vmem(degz, STRIPE * DEG_W)
        pltpu.sync_copy(degz, dacc.at[pl.ds(sid * STRIPE, STRIPE)])
        plsc.subcore_barrier()

        def step(j, _):
            _unpack_hi(pk, j, dbuf, 0)
            pltpu.sync_copy(ones_v, dacc.at[dbuf.at[0]], add=True)
            return 0

        lax.fori_loop(0, CHUNKS, step, 0)
        plsc.subcore_barrier()
        pltpu.sync_copy(dacc.at[pl.ds(sid * STRIPE, STRIPE)],
                        deg_hbm.at[cid, pl.ds(sid * STRIPE, STRIPE)])

    return pl.kernel(body, out_type=out_type, mesh=_MESH,
                     scratch_types=tuple(scratch),
                     compiler_params=pltpu.CompilerParams(
                         use_tc_tiling_on_sc=False))


_agg = _make_agg()
_deg = _make_deg()

_BLK = 1000
_GRID = N // _BLK


def _mm_body(x_ref, w_ref, o_ref):
    o_ref[...] = jnp.dot(x_ref[...], w_ref[...],
                         preferred_element_type=jnp.float32)


def _tc_matmul(x, w):
    return pl.pallas_call(
        _mm_body,
        grid=(_GRID,),
        in_specs=[pl.BlockSpec((_BLK, D), lambda i: (i, 0)),
                  pl.BlockSpec((D, D), lambda i: (0, 0))],
        out_specs=pl.BlockSpec((_BLK, D), lambda i: (i, 0)),
        out_shape=jax.ShapeDtypeStruct((N, D), jnp.float32),
    )(x, w)


def _neigh(p_ref, dg_ref):
    deg = jnp.maximum(dg_ref[0] + dg_ref[1], 1.0)
    inv = 1.0 / deg[:, 0:1]
    return (p_ref[0] + p_ref[1]) * inv


def _p_specs():
    return [pl.BlockSpec((NC, _BLK, D), lambda i: (0, i, 0)),
            pl.BlockSpec((NC, _BLK, DEG_W), lambda i: (0, i, 0))]


def _combine_body(h_ref, p_ref, dg_ref, ws_ref, b_ref, wn_ref, ho_ref, to_ref):
    h = (jnp.dot(h_ref[...], ws_ref[...], preferred_element_type=jnp.float32)
         + _neigh(p_ref, dg_ref) + b_ref[...])
    ho_ref[...] = h
    to_ref[...] = jnp.dot(h, wn_ref[...], preferred_element_type=jnp.float32)


def _tc_combine(h, p, dg, ws, b, wn):
    return pl.pallas_call(
        _combine_body,
        grid=(_GRID,),
        in_specs=[pl.BlockSpec((_BLK, D), lambda i: (i, 0))] + _p_specs() +
                 [pl.BlockSpec((D, D), lambda i: (0, 0)),
                  pl.BlockSpec((1, D), lambda i: (0, 0)),
                  pl.BlockSpec((D, D), lambda i: (0, 0))],
        out_specs=[pl.BlockSpec((_BLK, D), lambda i: (i, 0)),
                   pl.BlockSpec((_BLK, D), lambda i: (i, 0))],
        out_shape=[jax.ShapeDtypeStruct((N, D), jnp.float32),
                   jax.ShapeDtypeStruct((N, D), jnp.float32)],
    )(h, p, dg, ws, b.reshape(1, D), wn)


def _final_body(h_ref, p_ref, dg_ref, ws_ref, b_ref, ho_ref):
    h = (jnp.dot(h_ref[...], ws_ref[...], preferred_element_type=jnp.float32)
         + _neigh(p_ref, dg_ref) + b_ref[...])
    ho_ref[...] = jnp.maximum(h, 0.0)


def _tc_final(h, p, dg, ws, b):
    return pl.pallas_call(
        _final_body,
        grid=(_GRID,),
        in_specs=[pl.BlockSpec((_BLK, D), lambda i: (i, 0))] + _p_specs() +
                 [pl.BlockSpec((D, D), lambda i: (0, 0)),
                  pl.BlockSpec((1, D), lambda i: (0, 0))],
        out_specs=pl.BlockSpec((_BLK, D), lambda i: (i, 0)),
        out_shape=jax.ShapeDtypeStruct((N, D), jnp.float32),
    )(h, p, dg, ws, b.reshape(1, D))


def kernel(x, edge_index, W_self1, W_neigh1, b1, W_self2, W_neigh2, b2,
           W_self3, W_neigh3, b3):
    src = edge_index[0]
    dst = edge_index[1]
    pad = E_PAD - NUM_EDGES
    ar = jnp.arange(pad, dtype=jnp.int32)
    dummy_dst = N + (ar % (N_PAD - N))
    dummy_src = (ar * 13) % N
    src_p = jnp.concatenate([src, dummy_src])
    dst_p = jnp.concatenate([dst, dummy_dst])
    pk = ((dst_p << 16) | src_p).reshape(NW, CHUNKS, CHUNK)

    dg = _deg(pk)

    t1 = _tc_matmul(x, W_neigh1)
    h1, t2 = _tc_combine(x, _agg(t1, pk), dg, W_self1, b1, W_neigh2)
    h2, t3 = _tc_combine(h1, _agg(t2, pk), dg, W_self2, b2, W_neigh3)
    return _tc_final(h2, _agg(t3, pk), dg, W_self3, b3)


# speedup vs baseline: 1.2665x; 1.2665x over previous
import jax
import jax.numpy as jnp
from jax import lax
from jax.experimental import pallas as pl
from jax.experimental.pallas import tpu as pltpu
from jax.experimental.pallas import tpu_sc as plsc

N = 10000
D = 128
NUM_EDGES = 320000

NC = 2
NS = 16
NW = NC * NS

CHUNK = 128
CHUNKS = 80
HALF = CHUNKS // 2
E_PAD = NW * CHUNKS * CHUNK
N_PAD = 10240
STRIPE = N_PAD // NS
DEG_W = 16

_MESH = plsc.VectorSubcoreMesh(core_axis_name="c", subcore_axis_name="s")


def _zero_vmem(ref, nwords):
    cols = ref.shape[-1]
    per_row = cols // 16

    def body(i, _):
        r = lax.div(i, per_row)
        c = lax.rem(i, per_row) * 16
        ref[r, pl.ds(c, 16)] = jnp.zeros((16,), jnp.float32)
        return 0

    lax.fori_loop(0, nwords // 16, body, 0)


def _fill_ones(ref):
    rows, cols = ref.shape
    per_row = cols // 16

    def body(i, _):
        r = lax.div(i, per_row)
        c = lax.rem(i, per_row) * 16
        ref[r, pl.ds(c, 16)] = jnp.ones((16,), jnp.float32)
        return 0

    lax.fori_loop(0, rows * per_row, body, 0)


def _unpack_lo(pk, j, buf, b):
    def f(k, _):
        v = pk[j, pl.ds(k * 16, 16)]
        buf[b, pl.ds(k * 16, 16)] = jnp.bitwise_and(v, 0xFFFF)
        return 0
    lax.fori_loop(0, CHUNK // 16, f, 0)


def _unpack_hi(pk, j, buf, b):
    def f(k, _):
        v = pk[j, pl.ds(k * 16, 16)]
        buf[b, pl.ds(k * 16, 16)] = jnp.right_shift(v, 16)
        return 0
    lax.fori_loop(0, CHUNK // 16, f, 0)


def _make_agg():
    out_type = jax.ShapeDtypeStruct((NC, N_PAD, D), jnp.float32)
    scratch = [
        pltpu.VMEM((CHUNKS, CHUNK), jnp.int32),
        pltpu.VMEM((2, CHUNK), jnp.int32),
        pltpu.VMEM((2, CHUNK), jnp.int32),
        pltpu.VMEM((2, CHUNK, D), jnp.float32),
        pltpu.VMEM_SHARED((N_PAD, D), jnp.float32),
        pltpu.SemaphoreType.DMA,
        pltpu.SemaphoreType.DMA,
        pltpu.SemaphoreType.DMA,
        pltpu.SemaphoreType.DMA,
    ]

    def body(t_hbm, pk_hbm, out_hbm, pk, sbuf, dbuf, rows, acc,
             gsem0, gsem1, ssem0, ssem1):
        cid = lax.axis_index("c")
        sid = lax.axis_index("s")
        w = cid * NS + sid

        pltpu.sync_copy(pk_hbm.at[w], pk)

        _zero_vmem(rows.at[0], CHUNK * D)
        for k in range(STRIPE // CHUNK):
            pltpu.sync_copy(rows.at[0], acc.at[pl.ds(sid * STRIPE + k * CHUNK, CHUNK)])
        plsc.subcore_barrier()

        _unpack_lo(pk, 0, sbuf, 0)
        pltpu.async_copy(t_hbm.at[sbuf.at[0]], rows.at[0], gsem0)

        def step(i, _):
            j0 = 2 * i
            j1 = j0 + 1
            _unpack_lo(pk, j1, sbuf, 1)
            pltpu.async_copy(t_hbm.at[sbuf.at[1]], rows.at[1], gsem1)
            pltpu.make_async_copy(t_hbm.at[sbuf.at[0]], rows.at[0], gsem0).wait()
            _unpack_hi(pk, j0, dbuf, 0)
            pltpu.sync_copy(rows.at[0], acc.at[dbuf.at[0]], add=True)

            @pl.when(i + 1 < HALF)
            def _():
                _unpack_lo(pk, j0 + 2, sbuf, 0)
                pltpu.async_copy(t_hbm.at[sbuf.at[0]], rows.at[0], gsem0)

            pltpu.make_async_copy(t_hbm.at[sbuf.at[1]], rows.at[1], gsem1).wait()
            _unpack_hi(pk, j1, dbuf, 1)
            pltpu.sync_copy(rows.at[1], acc.at[dbuf.at[1]], add=True)
            return 0

        lax.fori_loop(0, HALF, step, 0)
        plsc.subcore_barrier()

        pltpu.sync_copy(acc.at[pl.ds(sid * STRIPE, STRIPE)],
                        out_hbm.at[cid, pl.ds(sid * STRIPE, STRIPE)])

    return pl.kernel(body, out_type=out_type, mesh=_MESH,
                     scratch_types=tuple(scratch),
                     compiler_params=pltpu.CompilerParams(
                         use_tc_tiling_on_sc=False))


def _make_deg():
    out_type = jax.ShapeDtypeStruct((NC, N_PAD, DEG_W), jnp.float32)
    scratch = [
        pltpu.VMEM((CHUNKS, CHUNK), jnp.int32),
        pltpu.VMEM((1, CHUNK), jnp.int32),
        pltpu.VMEM((CHUNK, DEG_W), jnp.float32),
        pltpu.VMEM((STRIPE, DEG_W), jnp.float32),
        pltpu.VMEM_SHARED((N_PAD, DEG_W), jnp.float32),
    ]

    def body(pk_hbm, deg_hbm, pk, dbuf, ones_v, degz, dacc):
        cid = lax.axis_index("c")
        sid = lax.axis_index("s")
        w = cid * NS + sid

        pltpu.sync_copy(pk_hbm.at[w], pk)
        _fill_ones(ones_v)
        _zero_vmem(degz, STRIPE * DEG_W)
        pltpu.sync_copy(degz, dacc.at[pl.ds(sid * STRIPE, STRIPE)])
        plsc.subcore_barrier()

        def step(j, _):
            _unpack_hi(pk, j, dbuf, 0)
            pltpu.sync_copy(ones_v, dacc.at[dbuf.at[0]], add=True)
            return 0

        lax.fori_loop(0, CHUNKS, step, 0)
        plsc.subcore_barrier()
        pltpu.sync_copy(dacc.at[pl.ds(sid * STRIPE, STRIPE)],
                        deg_hbm.at[cid, pl.ds(sid * STRIPE, STRIPE)])

    return pl.kernel(body, out_type=out_type, mesh=_MESH,
                     scratch_types=tuple(scratch),
                     compiler_params=pltpu.CompilerParams(
                         use_tc_tiling_on_sc=False))


_agg = _make_agg()
_deg = _make_deg()

_BLK = 1000
_GRID = N // _BLK


def _mm_body(x_ref, w_ref, o_ref):
    o_ref[...] = jnp.dot(x_ref[...], w_ref[...],
                         preferred_element_type=jnp.float32)


def _tc_matmul(x, w):
    return pl.pallas_call(
        _mm_body,
        grid=(_GRID,),
        in_specs=[pl.BlockSpec((_BLK, D), lambda i: (i, 0)),
                  pl.BlockSpec((D, D), lambda i: (0, 0))],
        out_specs=pl.BlockSpec((_BLK, D), lambda i: (i, 0)),
        out_shape=jax.ShapeDtypeStruct((N, D), jnp.float32),
    )(x, w)


def _neigh(p_ref, dg_ref):
    deg = jnp.maximum(dg_ref[0] + dg_ref[1], 1.0)
    inv = 1.0 / deg[:, 0:1]
    return (p_ref[0] + p_ref[1]) * inv


def _p_specs():
    return [pl.BlockSpec((NC, _BLK, D), lambda i: (0, i, 0)),
            pl.BlockSpec((NC, _BLK, DEG_W), lambda i: (0, i, 0))]


def _combine_body(h_ref, p_ref, dg_ref, ws_ref, b_ref, wn_ref, ho_ref, to_ref):
    h = (jnp.dot(h_ref[...], ws_ref[...], preferred_element_type=jnp.float32)
         + _neigh(p_ref, dg_ref) + b_ref[...])
    ho_ref[...] = h
    to_ref[...] = jnp.dot(h, wn_ref[...], preferred_element_type=jnp.float32)


def _tc_combine(h, p, dg, ws, b, wn):
    return pl.pallas_call(
        _combine_body,
        grid=(_GRID,),
        in_specs=[pl.BlockSpec((_BLK, D), lambda i: (i, 0))] + _p_specs() +
                 [pl.BlockSpec((D, D), lambda i: (0, 0)),
                  pl.BlockSpec((1, D), lambda i: (0, 0)),
                  pl.BlockSpec((D, D), lambda i: (0, 0))],
        out_specs=[pl.BlockSpec((_BLK, D), lambda i: (i, 0)),
                   pl.BlockSpec((_BLK, D), lambda i: (i, 0))],
        out_shape=[jax.ShapeDtypeStruct((N, D), jnp.float32),
                   jax.ShapeDtypeStruct((N, D), jnp.float32)],
    )(h, p, dg, ws, b.reshape(1, D), wn)


def _final_body(h_ref, p_ref, dg_ref, ws_ref, b_ref, ho_ref):
    h = (jnp.dot(h_ref[...], ws_ref[...], preferred_element_type=jnp.float32)
         + _neigh(p_ref, dg_ref) + b_ref[...])
    ho_ref[...] = jnp.maximum(h, 0.0)


def _tc_final(h, p, dg, ws, b):
    return pl.pallas_call(
        _final_body,
        grid=(_GRID,),
        in_specs=[pl.BlockSpec((_BLK, D), lambda i: (i, 0))] + _p_specs() +
                 [pl.BlockSpec((D, D), lambda i: (0, 0)),
                  pl.BlockSpec((1, D), lambda i: (0, 0))],
        out_specs=pl.BlockSpec((_BLK, D), lambda i: (i, 0)),
        out_shape=jax.ShapeDtypeStruct((N, D), jnp.float32),
    )(h, p, dg, ws, b.reshape(1, D))


def kernel(x, edge_index, W_self1, W_neigh1, b1, W_self2, W_neigh2, b2,
           W_self3, W_neigh3, b3):
    src = edge_index[0]
    dst = edge_index[1]
    pad = E_PAD - NUM_EDGES
    ar = jnp.arange(pad, dtype=jnp.int32)
    dummy_dst = N + (ar % (N_PAD - N))
    dummy_src = (ar * 13) % N
    src_p = jnp.concatenate([src, dummy_src])
    dst_p = jnp.concatenate([dst, dummy_dst])
    pk = ((dst_p << 16) | src_p).reshape(NW, CHUNKS, CHUNK)

    dg = _deg(pk)

    t1 = _tc_matmul(x, W_neigh1)
    h1, t2 = _tc_combine(x, _agg(t1, pk), dg, W_self1, b1, W_neigh2)
    h2, t3 = _tc_combine(h1, _agg(t2, pk), dg, W_self2, b2, W_neigh3)
    return _tc_final(h2, _agg(t3, pk), dg, W_self3, b3)


# speedup vs baseline: 1.3804x; 1.0899x over previous
import jax
import jax.numpy as jnp
from jax import lax
from jax.experimental import pallas as pl
from jax.experimental.pallas import tpu as pltpu
from jax.experimental.pallas import tpu_sc as plsc

N = 10000
D = 128
NUM_EDGES = 320000

NC = 2
NS = 16
NW = NC * NS

CHUNK = 128
CHUNKS = 80
HALF = CHUNKS // 2
E_PAD = NW * CHUNKS * CHUNK
N_PAD = 10240
STRIPE = N_PAD // NS
DEG_W = 16

_MESH = plsc.VectorSubcoreMesh(core_axis_name="c", subcore_axis_name="s")


def _zero_vmem(ref, nwords):
    cols = ref.shape[-1]
    per_row = cols // 16

    def body(i, _):
        r = lax.div(i, per_row)
        c = lax.rem(i, per_row) * 16
        ref[r, pl.ds(c, 16)] = jnp.zeros((16,), jnp.float32)
        return 0

    lax.fori_loop(0, nwords // 16, body, 0)


def _zero_vmem_bf16(ref, total):
    cols = ref.shape[-1]
    per_row = cols // 32

    def body(i, _):
        r = lax.div(i, per_row)
        c = lax.rem(i, per_row) * 32
        ref[r, pl.ds(c, 32)] = jnp.zeros((32,), jnp.bfloat16)
        return 0

    lax.fori_loop(0, total // 32, body, 0)


def _fill_ones(ref):
    rows, cols = ref.shape
    per_row = cols // 16

    def body(i, _):
        r = lax.div(i, per_row)
        c = lax.rem(i, per_row) * 16
        ref[r, pl.ds(c, 16)] = jnp.ones((16,), jnp.float32)
        return 0

    lax.fori_loop(0, rows * per_row, body, 0)


def _unpack_lo(pk, j, buf, b):
    def f(k, _):
        v = pk[j, pl.ds(k * 16, 16)]
        buf[b, pl.ds(k * 16, 16)] = jnp.bitwise_and(v, 0xFFFF)
        return 0
    lax.fori_loop(0, CHUNK // 16, f, 0)


def _unpack_hi(pk, j, buf, b):
    def f(k, _):
        v = pk[j, pl.ds(k * 16, 16)]
        buf[b, pl.ds(k * 16, 16)] = jnp.right_shift(v, 16)
        return 0
    lax.fori_loop(0, CHUNK // 16, f, 0)


def _make_agg():
    out_type = jax.ShapeDtypeStruct((NC, N_PAD, D), jnp.bfloat16)
    scratch = [
        pltpu.VMEM((CHUNKS, CHUNK), jnp.int32),
        pltpu.VMEM((2, CHUNK), jnp.int32),
        pltpu.VMEM((2, CHUNK), jnp.int32),
        pltpu.VMEM((2, CHUNK, D), jnp.bfloat16),
        pltpu.VMEM_SHARED((N_PAD, D), jnp.bfloat16),
        pltpu.SemaphoreType.DMA,
        pltpu.SemaphoreType.DMA,
    ]

    def body(t_hbm, pk_hbm, out_hbm, pk, sbuf, dbuf, rows, acc,
             gsem0, gsem1):
        cid = lax.axis_index("c")
        sid = lax.axis_index("s")
        w = cid * NS + sid

        pltpu.sync_copy(pk_hbm.at[w], pk)

        _zero_vmem_bf16(rows.at[0], CHUNK * D)
        for k in range(STRIPE // CHUNK):
            pltpu.sync_copy(rows.at[0], acc.at[pl.ds(sid * STRIPE + k * CHUNK, CHUNK)])
        plsc.subcore_barrier()

        _unpack_lo(pk, 0, sbuf, 0)
        pltpu.async_copy(t_hbm.at[sbuf.at[0]], rows.at[0], gsem0)

        def step(i, _):
            j0 = 2 * i
            j1 = j0 + 1
            _unpack_lo(pk, j1, sbuf, 1)
            pltpu.async_copy(t_hbm.at[sbuf.at[1]], rows.at[1], gsem1)
            pltpu.make_async_copy(t_hbm.at[sbuf.at[0]], rows.at[0], gsem0).wait()
            _unpack_hi(pk, j0, dbuf, 0)
            pltpu.sync_copy(rows.at[0], acc.at[dbuf.at[0]], add=True)

            @pl.when(i + 1 < HALF)
            def _():
                _unpack_lo(pk, j0 + 2, sbuf, 0)
                pltpu.async_copy(t_hbm.at[sbuf.at[0]], rows.at[0], gsem0)

            pltpu.make_async_copy(t_hbm.at[sbuf.at[1]], rows.at[1], gsem1).wait()
            _unpack_hi(pk, j1, dbuf, 1)
            pltpu.sync_copy(rows.at[1], acc.at[dbuf.at[1]], add=True)
            return 0

        lax.fori_loop(0, HALF, step, 0)
        plsc.subcore_barrier()

        pltpu.sync_copy(acc.at[pl.ds(sid * STRIPE, STRIPE)],
                        out_hbm.at[cid, pl.ds(sid * STRIPE, STRIPE)])

    return pl.kernel(body, out_type=out_type, mesh=_MESH,
                     scratch_types=tuple(scratch),
                     compiler_params=pltpu.CompilerParams(
                         use_tc_tiling_on_sc=False))


def _make_deg():
    out_type = jax.ShapeDtypeStruct((NC, N_PAD, DEG_W), jnp.float32)
    scratch = [
        pltpu.VMEM((CHUNKS, CHUNK), jnp.int32),
        pltpu.VMEM((1, CHUNK), jnp.int32),
        pltpu.VMEM((CHUNK, DEG_W), jnp.float32),
        pltpu.VMEM((STRIPE, DEG_W), jnp.float32),
        pltpu.VMEM_SHARED((N_PAD, DEG_W), jnp.float32),
    ]

    def body(pk_hbm, deg_hbm, pk, dbuf, ones_v, degz, dacc):
        cid = lax.axis_index("c")
        sid = lax.axis_index("s")
        w = cid * NS + sid

        pltpu.sync_copy(pk_hbm.at[w], pk)
        _fill_ones(ones_v)
        _zero_vmem(degz, STRIPE * DEG_W)
        pltpu.sync_copy(degz, dacc.at[pl.ds(sid * STRIPE, STRIPE)])
        plsc.subcore_barrier()

        def step(j, _):
            _unpack_hi(pk, j, dbuf, 0)
            pltpu.sync_copy(ones_v, dacc.at[dbuf.at[0]], add=True)
            return 0

        lax.fori_loop(0, CHUNKS, step, 0)
        plsc.subcore_barrier()
        pltpu.sync_copy(dacc.at[pl.ds(sid * STRIPE, STRIPE)],
                        deg_hbm.at[cid, pl.ds(sid * STRIPE, STRIPE)])

    return pl.kernel(body, out_type=out_type, mesh=_MESH,
                     scratch_types=tuple(scratch),
                     compiler_params=pltpu.CompilerParams(
                         use_tc_tiling_on_sc=False))


_agg = _make_agg()
_deg = _make_deg()

_BLK = 1000
_GRID = N // _BLK


def _mm_body(x_ref, w_ref, o_ref):
    o_ref[...] = jnp.dot(x_ref[...], w_ref[...],
                         preferred_element_type=jnp.float32
                         ).astype(jnp.bfloat16)


def _tc_matmul(x, w):
    return pl.pallas_call(
        _mm_body,
        grid=(_GRID,),
        in_specs=[pl.BlockSpec((_BLK, D), lambda i: (i, 0)),
                  pl.BlockSpec((D, D), lambda i: (0, 0))],
        out_specs=pl.BlockSpec((_BLK, D), lambda i: (i, 0)),
        out_shape=jax.ShapeDtypeStruct((N, D), jnp.bfloat16),
    )(x, w)


def _neigh(p_ref, dg_ref):
    deg = jnp.maximum(dg_ref[0] + dg_ref[1], 1.0)
    inv = 1.0 / deg[:, 0:1]
    return (p_ref[0].astype(jnp.float32) + p_ref[1].astype(jnp.float32)) * inv


def _p_specs():
    return [pl.BlockSpec((NC, _BLK, D), lambda i: (0, i, 0)),
            pl.BlockSpec((NC, _BLK, DEG_W), lambda i: (0, i, 0))]


def _combine_body(h_ref, p_ref, dg_ref, ws_ref, b_ref, wn_ref, ho_ref, to_ref):
    h = (jnp.dot(h_ref[...], ws_ref[...], preferred_element_type=jnp.float32)
         + _neigh(p_ref, dg_ref) + b_ref[...])
    ho_ref[...] = h
    to_ref[...] = jnp.dot(h, wn_ref[...], preferred_element_type=jnp.float32
                          ).astype(jnp.bfloat16)


def _tc_combine(h, p, dg, ws, b, wn):
    return pl.pallas_call(
        _combine_body,
        grid=(_GRID,),
        in_specs=[pl.BlockSpec((_BLK, D), lambda i: (i, 0))] + _p_specs() +
                 [pl.BlockSpec((D, D), lambda i: (0, 0)),
                  pl.BlockSpec((1, D), lambda i: (0, 0)),
                  pl.BlockSpec((D, D), lambda i: (0, 0))],
        out_specs=[pl.BlockSpec((_BLK, D), lambda i: (i, 0)),
                   pl.BlockSpec((_BLK, D), lambda i: (i, 0))],
        out_shape=[jax.ShapeDtypeStruct((N, D), jnp.float32),
                   jax.ShapeDtypeStruct((N, D), jnp.bfloat16)],
    )(h, p, dg, ws, b.reshape(1, D), wn)


def _final_body(h_ref, p_ref, dg_ref, ws_ref, b_ref, ho_ref):
    h = (jnp.dot(h_ref[...], ws_ref[...], preferred_element_type=jnp.float32)
         + _neigh(p_ref, dg_ref) + b_ref[...])
    ho_ref[...] = jnp.maximum(h, 0.0)


def _tc_final(h, p, dg, ws, b):
    return pl.pallas_call(
        _final_body,
        grid=(_GRID,),
        in_specs=[pl.BlockSpec((_BLK, D), lambda i: (i, 0))] + _p_specs() +
                 [pl.BlockSpec((D, D), lambda i: (0, 0)),
                  pl.BlockSpec((1, D), lambda i: (0, 0))],
        out_specs=pl.BlockSpec((_BLK, D), lambda i: (i, 0)),
        out_shape=jax.ShapeDtypeStruct((N, D), jnp.float32),
    )(h, p, dg, ws, b.reshape(1, D))


def kernel(x, edge_index, W_self1, W_neigh1, b1, W_self2, W_neigh2, b2,
           W_self3, W_neigh3, b3):
    src = edge_index[0]
    dst = edge_index[1]
    pad = E_PAD - NUM_EDGES
    ar = jnp.arange(pad, dtype=jnp.int32)
    dummy_dst = N + (ar % (N_PAD - N))
    dummy_src = (ar * 13) % N
    src_p = jnp.concatenate([src, dummy_src])
    dst_p = jnp.concatenate([dst, dummy_dst])
    pk = ((dst_p << 16) | src_p).reshape(NW, CHUNKS, CHUNK)

    dg = _deg(pk)

    t1 = _tc_matmul(x, W_neigh1)
    h1, t2 = _tc_combine(x, _agg(t1, pk), dg, W_self1, b1, W_neigh2)
    h2, t3 = _tc_combine(h1, _agg(t2, pk), dg, W_self2, b2, W_neigh3)
    return _tc_final(h2, _agg(t3, pk), dg, W_self3, b3)
